# Initial kernel scaffold; baseline (speedup 1.0000x reference)
import functools

import jax
import jax.numpy as jnp
from jax import lax
from jax.experimental import pallas as pl
from jax.experimental.pallas import tpu as pltpu
from jax.experimental.pallas import tpu_sc as plsc

S = 4
K5 = 5
SS = S * S
KK = K5 * K5
D = SS * KK


def _argmax_body(keys_ref, q_ref, out_ref, *, n):
    q = q_ref[0]
    sim = lax.dot_general(
        keys_ref[...], q,
        dimension_numbers=(((1,), (0,)), ((), ())),
        preferred_element_type=jnp.float32,
        precision=lax.Precision.HIGHEST,
    )
    m = jnp.max(sim, axis=0, keepdims=True)
    rid = lax.broadcasted_iota(jnp.int32, sim.shape, 0)
    idx = jnp.min(jnp.where(sim == m, rid, n), axis=0)
    out_ref[0, 0, :] = idx


def _argmax_idx(queries, keys, p_tile=512):
    B, L, H, W = queries.shape
    n = keys.shape[0]
    hw = H * W
    nb = hw // p_tile
    qf = queries.reshape(B, L, hw)
    grid = (B * nb,)
    out = pl.pallas_call(
        functools.partial(_argmax_body, n=n),
        grid=grid,
        in_specs=[
            pl.BlockSpec((n, L), lambda g: (0, 0)),
            pl.BlockSpec((1, L, p_tile), lambda g, _nb=nb: (g // _nb, 0, g % _nb)),
        ],
        out_specs=pl.BlockSpec((1, 1, p_tile), lambda g: (g, 0, 0)),
        out_shape=jax.ShapeDtypeStruct((B * nb, 1, p_tile), jnp.int32),
    )(keys, qf)
    return out.reshape(B * hw)


def _make_sc_gather(n_rows, d, b_tot):
    info = plsc.get_sparse_core_info()
    nw = info.num_cores * info.num_subcores
    b_per_w = b_tot // nw
    ch = 224
    n_chunks = b_per_w // ch
    assert b_per_w % ch == 0 and b_per_w % 8 == 0

    mesh = plsc.VectorSubcoreMesh(core_axis_name="c", subcore_axis_name="s")

    @functools.partial(
        pl.kernel,
        mesh=mesh,
        out_type=jax.ShapeDtypeStruct((b_tot, d), jnp.float32),
        scratch_types=[
            pltpu.VMEM((b_per_w,), jnp.int32),
            pltpu.VMEM((ch, d), jnp.float32),
            pltpu.SemaphoreType.DMA,
        ],
    )
    def gather_k(table_hbm, idx_hbm, out_hbm, idx_v, rows_v, sem):
        wid = lax.axis_index("s") * info.num_cores + lax.axis_index("c")
        base = wid * b_per_w
        pltpu.sync_copy(idx_hbm.at[pl.ds(base, b_per_w)], idx_v)

        def body(i, carry):
            off = i * ch
            pltpu.async_copy(
                table_hbm.at[idx_v.at[pl.ds(off, ch)]], rows_v, sem
            ).wait()
            pltpu.sync_copy(rows_v, out_hbm.at[pl.ds(base + off, ch)])
            return carry

        lax.fori_loop(0, n_chunks, body, 0)

    return gather_k


def _conv_body(xp_ref, g_ref, out_ref, *, rh, h_blocks, w):
    gidx = pl.program_id(0)
    b = gidx // h_blocks
    h0 = (gidx % h_blocks) * rh
    gt = jnp.transpose(g_ref[...], (2, 0, 1))
    for c in range(3):
        xs = [
            xp_ref[b, c, pl.ds(h0 + i5, rh), pl.ds(j5, w)]
            for i5 in range(K5) for j5 in range(K5)
        ]
        for ss in range(SS):
            acc = xs[0] * gt[ss * KK]
            for k in range(1, KK):
                acc = acc + xs[k] * gt[ss * KK + k]
            out_ref[0, c, ss] = acc


def _patch_conv(x_pad, gathered, B, C, H, W, rh=8):
    h_blocks = H // rh
    g3 = gathered.reshape(B * H, W, D)
    grid = (B * h_blocks,)
    out5 = pl.pallas_call(
        functools.partial(_conv_body, rh=rh, h_blocks=h_blocks, w=W),
        grid=grid,
        in_specs=[
            pl.BlockSpec(x_pad.shape, lambda g: (0, 0, 0, 0)),
            pl.BlockSpec((rh, W, D), lambda g: (g, 0, 0)),
        ],
        out_specs=pl.BlockSpec(
            (1, C, SS, rh, W),
            lambda g, _hb=h_blocks: (g // _hb, 0, 0, g % _hb, 0),
        ),
        out_shape=jax.ShapeDtypeStruct((B, C, SS, H, W), jnp.float32),
    )(x_pad, g3)
    return out5


def kernel(x, queries, keys, values):
    B, C, H, W = x.shape
    n = keys.shape[0]
    pad = K5 // 2

    idx = _argmax_idx(queries, keys)

    table = values.reshape(n, D)
    gathered = _make_sc_gather(n, D, B * H * W)(table, idx)

    x_pad = jnp.pad(x, ((0, 0), (0, 0), (pad, pad), (pad, pad)),
                    mode="reflect")
    out5 = _patch_conv(x_pad, gathered, B, C, H, W)

    out = out5.reshape(B, C, S, S, H, W)
    out = jnp.transpose(out, (0, 1, 4, 2, 5, 3)).reshape(B, C, H * S, W * S)
    return out


# speedup vs baseline: 2.7163x; 2.7163x over previous
import functools

import jax
import jax.numpy as jnp
from jax import lax
from jax.experimental import pallas as pl
from jax.experimental.pallas import tpu as pltpu
from jax.experimental.pallas import tpu_sc as plsc

S = 4
K5 = 5
SS = S * S
KK = K5 * K5
D = SS * KK


def _argmax_body(keys_ref, q_ref, out_ref, *, n):
    q = q_ref[0].astype(jnp.bfloat16)
    sim = lax.dot_general(
        keys_ref[...].astype(jnp.bfloat16), q,
        dimension_numbers=(((1,), (0,)), ((), ())),
        preferred_element_type=jnp.float32,
    )
    m = jnp.max(sim, axis=0, keepdims=True)
    rid = lax.broadcasted_iota(jnp.int32, sim.shape, 0)
    idx = jnp.min(jnp.where(sim == m, rid, n), axis=0)
    out_ref[0, 0, :] = idx


def _argmax_idx(queries, keys, p_tile=512):
    B, L, H, W = queries.shape
    n = keys.shape[0]
    hw = H * W
    nb = hw // p_tile
    qf = queries.reshape(B, L, hw)
    grid = (B * nb,)
    out = pl.pallas_call(
        functools.partial(_argmax_body, n=n),
        grid=grid,
        in_specs=[
            pl.BlockSpec((n, L), lambda g: (0, 0)),
            pl.BlockSpec((1, L, p_tile), lambda g, _nb=nb: (g // _nb, 0, g % _nb)),
        ],
        out_specs=pl.BlockSpec((1, 1, p_tile), lambda g: (g, 0, 0)),
        out_shape=jax.ShapeDtypeStruct((B * nb, 1, p_tile), jnp.int32),
    )(keys, qf)
    return out.reshape(B * hw)


def _make_sc_gather(n_rows, d, b_tot):
    info = plsc.get_sparse_core_info()
    nw = info.num_cores * info.num_subcores
    b_per_w = b_tot // nw
    ch = 112
    n_chunks = b_per_w // ch
    assert b_per_w % ch == 0 and b_per_w % 8 == 0

    mesh = plsc.VectorSubcoreMesh(core_axis_name="c", subcore_axis_name="s")

    @functools.partial(
        pl.kernel,
        mesh=mesh,
        out_type=jax.ShapeDtypeStruct((b_tot, d), jnp.float32),
        scratch_types=[
            pltpu.VMEM((b_per_w,), jnp.int32),
            pltpu.VMEM((ch, d), jnp.float32),
            pltpu.SemaphoreType.DMA,
        ],
    )
    def gather_k(table_hbm, idx_hbm, out_hbm, idx_v, rows_v, sem):
        wid = lax.axis_index("s") * info.num_cores + lax.axis_index("c")
        base = wid * b_per_w
        pltpu.sync_copy(idx_hbm.at[pl.ds(base, b_per_w)], idx_v)

        def body(i, carry):
            off = i * ch
            pltpu.async_copy(
                table_hbm.at[idx_v.at[pl.ds(off, ch)]], rows_v, sem
            ).wait()
            pltpu.sync_copy(rows_v, out_hbm.at[pl.ds(base + off, ch)])
            return carry

        lax.fori_loop(0, n_chunks, body, 0)

    return gather_k


def _conv_body(xp_ref, g_ref, out_ref, *, rh, h_blocks, w):
    gidx = pl.program_id(0)
    b = gidx // h_blocks
    h0 = pl.multiple_of((gidx % h_blocks) * rh, rh)
    gt = jnp.transpose(g_ref[...], (2, 0, 1))
    for c in range(3):
        xblk = xp_ref[b, c, pl.ds(h0, rh + 8), :]
        xs = [
            xblk[i5:i5 + rh, j5:j5 + w]
            for i5 in range(K5) for j5 in range(K5)
        ]
        for ss in range(SS):
            acc = xs[0] * gt[ss * KK]
            for k in range(1, KK):
                acc = acc + xs[k] * gt[ss * KK + k]
            out_ref[0, c, ss] = acc


def _patch_conv(x_pad, gathered, B, C, H, W, rh=8):
    h_blocks = H // rh
    gd = gathered.shape[-1]
    g3 = gathered.reshape(B * H, W, gd)
    grid = (B * h_blocks,)
    out5 = pl.pallas_call(
        functools.partial(_conv_body, rh=rh, h_blocks=h_blocks, w=W),
        grid=grid,
        in_specs=[
            pl.BlockSpec(x_pad.shape, lambda g: (0, 0, 0, 0)),
            pl.BlockSpec((rh, W, gd), lambda g: (g, 0, 0)),
        ],
        out_specs=pl.BlockSpec(
            (1, C, SS, rh, W),
            lambda g, _hb=h_blocks: (g // _hb, 0, 0, g % _hb, 0),
        ),
        out_shape=jax.ShapeDtypeStruct((B, C, SS, H, W), jnp.float32),
    )(x_pad, g3)
    return out5


def kernel(x, queries, keys, values):
    B, C, H, W = x.shape
    n = keys.shape[0]
    pad = K5 // 2

    idx = _argmax_idx(queries, keys)

    dpad = 512
    table = jnp.pad(values.reshape(n, D), ((0, 0), (0, dpad - D)))
    gathered = _make_sc_gather(n, dpad, B * H * W)(table, idx)

    x_pad = jnp.pad(x, ((0, 0), (0, 0), (pad, pad), (pad, pad)),
                    mode="reflect")
    x_pad = jnp.pad(x_pad, ((0, 0), (0, 0), (0, 4), (0, 0)))
    out5 = _patch_conv(x_pad, gathered, B, C, H, W)

    out = out5.reshape(B, C, S, S, H, W)
    out = jnp.transpose(out, (0, 1, 4, 2, 5, 3)).reshape(B, C, H * S, W * S)
    return out
